# Initial kernel scaffold; baseline (speedup 1.0000x reference)
import jax
import jax.numpy as jnp
from jax.experimental import pallas as pl


def kernel(x, edge_index, edge_attr, batch, Wn, bn_, We0, be0, Wr0, br0, g0, bt0, We1, be1, Wr1, br1, g1, bt1, We2, be2, Wr2, br2, g2, bt2, W1, b1, W2, b2):
    raise NotImplementedError("write your pallas kernel here")


# speedup vs baseline: 3.0439x; 3.0439x over previous
import functools

import jax
import jax.numpy as jnp
from jax import lax
from jax.experimental import pallas as pl
from jax.experimental.pallas import tpu as pltpu
from jax.experimental.pallas import tpu_sc as plsc

N = 10000
E = 160000
H = 16
G = 64
NODE_DIM = 128
EDGE_DIM = 16
EPS = 1e-5

NC = 2
NS = 16
NW = NC * NS
CK = 128
CPW = 40
EPW = CPW * CK
E_PAD = NW * EPW
N_SH = 10112
NPT = N_SH // NS

_mesh = plsc.VectorSubcoreMesh(core_axis_name="c", subcore_axis_name="s")
_sc_params = pltpu.CompilerParams(use_tc_tiling_on_sc=False)


@functools.partial(
    pl.kernel,
    out_type=jax.ShapeDtypeStruct((E_PAD, H), jnp.float32),
    mesh=_mesh,
    scratch_types=[
        pltpu.VMEM((CPW, CK), jnp.int32),
        pltpu.VMEM((EPW, H), jnp.float32),
        pltpu.SemaphoreType.DMA,
    ],
    compiler_params=_sc_params,
)
def _sc_gather(h_hbm, idx_hbm, out_hbm, idx_v, rows_v, sem):
    wid = lax.axis_index("c") * NS + lax.axis_index("s")
    pltpu.sync_copy(idx_hbm.at[pl.ds(wid * CPW, CPW)], idx_v)

    def body(j, _):
        pltpu.async_copy(
            h_hbm.at[idx_v.at[j]], rows_v.at[pl.ds(j * CK, CK)], sem
        ).wait()
        return 0

    lax.fori_loop(0, CPW, body, 0)
    pltpu.sync_copy(rows_v, out_hbm.at[pl.ds(wid * EPW, EPW)])


@functools.partial(
    pl.kernel,
    out_type=jax.ShapeDtypeStruct((NC, N_SH, H), jnp.float32),
    mesh=_mesh,
    scratch_types=[
        pltpu.VMEM((CPW, CK), jnp.int32),
        pltpu.VMEM((EPW, H), jnp.float32),
        pltpu.VMEM_SHARED((N_SH, H), jnp.float32),
        pltpu.SemaphoreType.DMA,
    ],
    compiler_params=_sc_params,
)
def _sc_scatter(rows_hbm, idx_hbm, zeros_hbm, out_hbm, idx_v, rows_v, acc, sem):
    cid = lax.axis_index("c")
    sid = lax.axis_index("s")
    wid = cid * NS + sid
    pltpu.sync_copy(zeros_hbm.at[pl.ds(sid * NPT, NPT)],
                    acc.at[pl.ds(sid * NPT, NPT)])
    pltpu.sync_copy(idx_hbm.at[pl.ds(wid * CPW, CPW)], idx_v)
    pltpu.sync_copy(rows_hbm.at[pl.ds(wid * EPW, EPW)], rows_v)
    plsc.subcore_barrier()

    def body(j, _):
        pltpu.sync_copy(rows_v.at[pl.ds(j * CK, CK)], acc.at[idx_v.at[j]],
                        add=True)
        return 0

    lax.fori_loop(0, CPW, body, 0)
    plsc.subcore_barrier()
    pltpu.sync_copy(acc.at[pl.ds(sid * NPT, NPT)],
                    out_hbm.at[cid, pl.ds(sid * NPT, NPT)])


@functools.partial(
    pl.kernel,
    out_type=jax.ShapeDtypeStruct((NC, N_SH, H), jnp.float32),
    mesh=_mesh,
    scratch_types=[
        pltpu.VMEM((CPW, CK), jnp.int32),
        pltpu.VMEM((CK, H), jnp.float32),
        pltpu.VMEM_SHARED((N_SH, H), jnp.float32),
        pltpu.SemaphoreType.DMA,
    ],
    compiler_params=_sc_params,
)
def _sc_degree(idx_hbm, zeros_hbm, out_hbm, idx_v, ones_v, acc, sem):
    cid = lax.axis_index("c")
    sid = lax.axis_index("s")
    wid = cid * NS + sid
    pltpu.sync_copy(zeros_hbm.at[pl.ds(sid * NPT, NPT)],
                    acc.at[pl.ds(sid * NPT, NPT)])
    pltpu.sync_copy(idx_hbm.at[pl.ds(wid * CPW, CPW)], idx_v)

    def fill(i, _):
        ones_v[i] = jnp.ones((H,), jnp.float32)
        return 0

    lax.fori_loop(0, CK, fill, 0)
    plsc.subcore_barrier()

    def body(j, _):
        pltpu.sync_copy(ones_v, acc.at[idx_v.at[j]], add=True)
        return 0

    lax.fori_loop(0, CPW, body, 0)
    plsc.subcore_barrier()
    pltpu.sync_copy(acc.at[pl.ds(sid * NPT, NPT)],
                    out_hbm.at[cid, pl.ds(sid * NPT, NPT)])


_RN = 2000


def _init_body(x_ref, wn_ref, bn_ref, d0_ref, d1_ref, h_ref, ig_ref):
    h_ref[...] = (
        jnp.dot(x_ref[...], wn_ref[...], preferred_element_type=jnp.float32)
        + bn_ref[0:1, :]
    )
    deg = jnp.maximum(d0_ref[...] + d1_ref[...], 1.0)
    ig_ref[...] = 1.0 / deg


def _tc_init(x, wn, bn_row, d0, d1):
    grid = N // _RN
    return pl.pallas_call(
        _init_body,
        grid=(grid,),
        in_specs=[
            pl.BlockSpec((_RN, NODE_DIM), lambda i: (i, 0)),
            pl.BlockSpec((NODE_DIM, H), lambda i: (0, 0)),
            pl.BlockSpec((8, H), lambda i: (0, 0)),
            pl.BlockSpec((_RN, H), lambda i: (i, 0)),
            pl.BlockSpec((_RN, H), lambda i: (i, 0)),
        ],
        out_specs=[
            pl.BlockSpec((_RN, H), lambda i: (i, 0)),
            pl.BlockSpec((_RN, H), lambda i: (i, 0)),
        ],
        out_shape=[
            jax.ShapeDtypeStruct((N, H), jnp.float32),
            jax.ShapeDtypeStruct((N, H), jnp.float32),
        ],
    )(x, wn, bn_row, d0, d1)


_RE = 2048


def _msg_body(hs_ref, ea_ref, wt_ref, bm_ref, r_ref, s_ref, o_ref):
    hs = hs_ref[...]
    b = jnp.dot(hs, wt_ref[...], preferred_element_type=jnp.float32)
    ear = jnp.dot(ea_ref[...], r_ref[...], preferred_element_type=jnp.float32)
    o_ref[...] = (
        jnp.dot(ear * b, s_ref[...], preferred_element_type=jnp.float32)
        + jnp.dot(hs, bm_ref[...], preferred_element_type=jnp.float32)
    )


def _tc_msg(hs, ea, wt, bm, r, s):
    grid = E_PAD // _RE
    return pl.pallas_call(
        _msg_body,
        grid=(grid,),
        in_specs=[
            pl.BlockSpec((_RE, H), lambda i: (i, 0)),
            pl.BlockSpec((_RE, H), lambda i: (i, 0)),
            pl.BlockSpec((H, H * H), lambda i: (0, 0)),
            pl.BlockSpec((H, H), lambda i: (0, 0)),
            pl.BlockSpec((H, H * H), lambda i: (0, 0)),
            pl.BlockSpec((H * H, H), lambda i: (0, 0)),
        ],
        out_specs=pl.BlockSpec((_RE, H), lambda i: (i, 0)),
        out_shape=jax.ShapeDtypeStruct((E_PAD, H), jnp.float32),
    )(hs, ea, wt, bm, r, s)


def _update_body(h_ref, a0_ref, a1_ref, ig_ref, wr_ref, v_ref, o_ref):
    h = h_ref[...]
    agg = (a0_ref[...] + a1_ref[...]) * ig_ref[...]
    pre = (agg + jnp.dot(h, wr_ref[...], preferred_element_type=jnp.float32))
    out = pre * v_ref[0:1, :] + v_ref[1:2, :]
    o_ref[...] = h + jnp.maximum(out, 0.0)


def _tc_update(h, a0, a1, ig, wr, vab):
    grid = N // _RN
    return pl.pallas_call(
        _update_body,
        grid=(grid,),
        in_specs=[
            pl.BlockSpec((_RN, H), lambda i: (i, 0)),
            pl.BlockSpec((_RN, H), lambda i: (i, 0)),
            pl.BlockSpec((_RN, H), lambda i: (i, 0)),
            pl.BlockSpec((_RN, H), lambda i: (i, 0)),
            pl.BlockSpec((H, H), lambda i: (0, 0)),
            pl.BlockSpec((8, H), lambda i: (0, 0)),
        ],
        out_specs=pl.BlockSpec((_RN, H), lambda i: (i, 0)),
        out_shape=jax.ShapeDtypeStruct((N, H), jnp.float32),
    )(h, a0, a1, ig, wr, vab)


def _pool_body(h_ref, b_ref, w1_ref, b1_ref, w2_ref, b2_ref, o_ref,
               acc_ref, cnt_ref):
    i = pl.program_id(0)

    @pl.when(i == 0)
    def _():
        acc_ref[...] = jnp.zeros_like(acc_ref)
        cnt_ref[...] = jnp.zeros_like(cnt_ref)

    h = h_ref[...]
    onehot = (b_ref[...] == lax.broadcasted_iota(jnp.int32, (_RN, G), 1))
    onehot = onehot.astype(jnp.float32)
    dn = (((0,), (0,)), ((), ()))
    acc_ref[...] += lax.dot_general(onehot, h, dn,
                                    preferred_element_type=jnp.float32)
    cnt_ref[...] += lax.dot_general(onehot, jnp.ones_like(h), dn,
                                    preferred_element_type=jnp.float32)

    @pl.when(i == pl.num_programs(0) - 1)
    def _():
        pooled = acc_ref[...] / jnp.maximum(cnt_ref[...], 1.0)
        z = jnp.maximum(
            jnp.dot(pooled, w1_ref[...], preferred_element_type=jnp.float32)
            + b1_ref[0:1, :], 0.0)
        o_ref[...] = (
            jnp.dot(z, w2_ref[...], preferred_element_type=jnp.float32)
            + b2_ref[0:1, 0:1]
        )


def _tc_pool(h, batch2, w1, b1_row, w2, b2_row):
    grid = N // _RN
    return pl.pallas_call(
        _pool_body,
        grid=(grid,),
        in_specs=[
            pl.BlockSpec((_RN, H), lambda i: (i, 0)),
            pl.BlockSpec((_RN, 1), lambda i: (i, 0)),
            pl.BlockSpec((H, 64), lambda i: (0, 0)),
            pl.BlockSpec((8, 64), lambda i: (0, 0)),
            pl.BlockSpec((64, 1), lambda i: (0, 0)),
            pl.BlockSpec((8, 8), lambda i: (0, 0)),
        ],
        out_specs=pl.BlockSpec((G, 1), lambda i: (0, 0)),
        out_shape=jax.ShapeDtypeStruct((G, 1), jnp.float32),
        scratch_shapes=[
            pltpu.VMEM((G, H), jnp.float32),
            pltpu.VMEM((G, H), jnp.float32),
        ],
    )(h, batch2, w1, b1_row, w2, b2_row)


def kernel(x, edge_index, edge_attr, batch, Wn, bn_, We0, be0, Wr0, br0, g0,
           bt0, We1, be1, Wr1, br1, g1, bt1, We2, be2, Wr2, br2, g2, bt2,
           W1, b1, W2, b2):
    f32 = jnp.float32
    src = jnp.concatenate(
        [edge_index[0], jnp.zeros((E_PAD - E,), jnp.int32)]).reshape(
            NW * CPW, CK)
    dst = jnp.concatenate(
        [edge_index[1], jnp.full((E_PAD - E,), N, jnp.int32)]).reshape(
            NW * CPW, CK)
    ea = jnp.concatenate(
        [edge_attr, jnp.zeros((E_PAD - E, EDGE_DIM), f32)], axis=0)
    zeros_sh = jnp.zeros((N_SH, H), f32)

    r_mat = jnp.kron(jnp.eye(H, dtype=f32), jnp.ones((1, H), f32))
    s_mat = jnp.kron(jnp.ones((H, 1), f32), jnp.eye(H, dtype=f32))

    scale = 1.0 / jnp.sqrt(1.0 + EPS)
    layers = []
    for (We, be, Wr, br, g, bt) in (
        (We0, be0, Wr0, br0, g0, bt0),
        (We1, be1, Wr1, br1, g1, bt1),
        (We2, be2, Wr2, br2, g2, bt2),
    ):
        wt = We.reshape(H, H, H).transpose(1, 0, 2).reshape(H, H * H)
        bm = be.reshape(H, H)
        va = g * scale
        vb = br * va + bt
        vab = jnp.zeros((8, H), f32).at[0].set(va).at[1].set(vb)
        layers.append((wt, bm, Wr, vab))

    degp = _sc_degree(dst, zeros_sh)
    h, ig = _tc_init(x, Wn, jnp.broadcast_to(bn_, (8, H)), degp[0, :N],
                     degp[1, :N])

    for (wt, bm, wr, vab) in layers:
        hs = _sc_gather(h, src)
        msg = _tc_msg(hs, ea, wt, bm, r_mat, s_mat)
        aggp = _sc_scatter(msg, dst, zeros_sh)
        h = _tc_update(h, aggp[0, :N], aggp[1, :N], ig, wr, vab)

    out = _tc_pool(h, batch[:, None], W1, jnp.broadcast_to(b1, (8, 64)), W2,
                   jnp.broadcast_to(b2, (8, 8)))
    return out[:, 0]


# speedup vs baseline: 3.3095x; 1.0873x over previous
import functools

import jax
import jax.numpy as jnp
from jax import lax
from jax.experimental import pallas as pl
from jax.experimental.pallas import tpu as pltpu
from jax.experimental.pallas import tpu_sc as plsc

N = 10000
E = 160000
H = 16
G = 64
NODE_DIM = 128
EDGE_DIM = 16
EPS = 1e-5

NC = 2
NS = 16
NW = NC * NS
CK = 128
CPW = 40
EPW = CPW * CK
KG = 8
E_PAD = NW * EPW
N_SH = 10112
NPT = N_SH // NS

_mesh = plsc.VectorSubcoreMesh(core_axis_name="c", subcore_axis_name="s")
_sc_params = pltpu.CompilerParams(use_tc_tiling_on_sc=False)


@functools.partial(
    pl.kernel,
    out_type=jax.ShapeDtypeStruct((E_PAD, H), jnp.float32),
    mesh=_mesh,
    scratch_types=[
        pltpu.VMEM((CPW, CK), jnp.int32),
        pltpu.VMEM((EPW, H), jnp.float32),
        pltpu.SemaphoreType.DMA,
    ],
    compiler_params=_sc_params,
)
def _sc_gather(h_hbm, idx_hbm, out_hbm, idx_v, rows_v, sem):
    wid = lax.axis_index("c") * NS + lax.axis_index("s")
    pltpu.sync_copy(idx_hbm.at[pl.ds(wid * CPW, CPW)], idx_v)

    def group(g, _):
        base = g * KG
        for t in range(KG):
            pltpu.async_copy(h_hbm.at[idx_v.at[base + t]],
                             rows_v.at[pl.ds((base + t) * CK, CK)], sem)
        for t in range(KG):
            pltpu.make_async_copy(h_hbm.at[idx_v.at[base + t]],
                                  rows_v.at[pl.ds((base + t) * CK, CK)],
                                  sem).wait()
        return 0

    lax.fori_loop(0, CPW // KG, group, 0)
    pltpu.sync_copy(rows_v, out_hbm.at[pl.ds(wid * EPW, EPW)])


@functools.partial(
    pl.kernel,
    out_type=jax.ShapeDtypeStruct((NC, N_SH, H), jnp.float32),
    mesh=_mesh,
    scratch_types=[
        pltpu.VMEM((CPW, CK), jnp.int32),
        pltpu.VMEM((EPW, H), jnp.float32),
        pltpu.VMEM_SHARED((N_SH, H), jnp.float32),
        pltpu.SemaphoreType.DMA,
    ],
    compiler_params=_sc_params,
)
def _sc_scatter(rows_hbm, idx_hbm, zeros_hbm, out_hbm, idx_v, rows_v, acc, sem):
    cid = lax.axis_index("c")
    sid = lax.axis_index("s")
    wid = cid * NS + sid
    pltpu.sync_copy(zeros_hbm.at[pl.ds(sid * NPT, NPT)],
                    acc.at[pl.ds(sid * NPT, NPT)])
    pltpu.sync_copy(idx_hbm.at[pl.ds(wid * CPW, CPW)], idx_v)
    pltpu.sync_copy(rows_hbm.at[pl.ds(wid * EPW, EPW)], rows_v)
    plsc.subcore_barrier()

    def group(g, _):
        base = g * KG
        for t in range(KG):
            pltpu.async_copy(rows_v.at[pl.ds((base + t) * CK, CK)],
                             acc.at[idx_v.at[base + t]], sem, add=True)
        for t in range(KG):
            pltpu.make_async_copy(rows_v.at[pl.ds((base + t) * CK, CK)],
                                  acc.at[idx_v.at[base + t]], sem).wait()
        return 0

    lax.fori_loop(0, CPW // KG, group, 0)
    plsc.subcore_barrier()
    pltpu.sync_copy(acc.at[pl.ds(sid * NPT, NPT)],
                    out_hbm.at[cid, pl.ds(sid * NPT, NPT)])


@functools.partial(
    pl.kernel,
    out_type=jax.ShapeDtypeStruct((NC, N_SH, H), jnp.float32),
    mesh=_mesh,
    scratch_types=[
        pltpu.VMEM((CPW, CK), jnp.int32),
        pltpu.VMEM((CK, H), jnp.float32),
        pltpu.VMEM_SHARED((N_SH, H), jnp.float32),
        pltpu.SemaphoreType.DMA,
    ],
    compiler_params=_sc_params,
)
def _sc_degree(idx_hbm, zeros_hbm, out_hbm, idx_v, ones_v, acc, sem):
    cid = lax.axis_index("c")
    sid = lax.axis_index("s")
    wid = cid * NS + sid
    pltpu.sync_copy(zeros_hbm.at[pl.ds(sid * NPT, NPT)],
                    acc.at[pl.ds(sid * NPT, NPT)])
    pltpu.sync_copy(idx_hbm.at[pl.ds(wid * CPW, CPW)], idx_v)

    def fill(i, _):
        ones_v[i] = jnp.ones((H,), jnp.float32)
        return 0

    lax.fori_loop(0, CK, fill, 0)
    plsc.subcore_barrier()

    def group(g, _):
        base = g * KG
        for t in range(KG):
            pltpu.async_copy(ones_v, acc.at[idx_v.at[base + t]], sem,
                             add=True)
        for t in range(KG):
            pltpu.make_async_copy(ones_v, acc.at[idx_v.at[base + t]],
                                  sem).wait()
        return 0

    lax.fori_loop(0, CPW // KG, group, 0)
    plsc.subcore_barrier()
    pltpu.sync_copy(acc.at[pl.ds(sid * NPT, NPT)],
                    out_hbm.at[cid, pl.ds(sid * NPT, NPT)])


_RN = 2000


def _init_body(x_ref, wn_ref, bn_ref, dp_ref, h_ref, ig_ref):
    h_ref[...] = (
        jnp.dot(x_ref[...], wn_ref[...], preferred_element_type=jnp.float32)
        + bn_ref[0:1, :]
    )
    deg = jnp.maximum(dp_ref[0] + dp_ref[1], 1.0)
    ig_ref[...] = 1.0 / deg


def _tc_init(x, wn, bn_row, degp):
    grid = N // _RN
    return pl.pallas_call(
        _init_body,
        grid=(grid,),
        in_specs=[
            pl.BlockSpec((_RN, NODE_DIM), lambda i: (i, 0)),
            pl.BlockSpec((NODE_DIM, H), lambda i: (0, 0)),
            pl.BlockSpec((8, H), lambda i: (0, 0)),
            pl.BlockSpec((NC, _RN, H), lambda i: (0, i, 0)),
        ],
        out_specs=[
            pl.BlockSpec((_RN, H), lambda i: (i, 0)),
            pl.BlockSpec((_RN, H), lambda i: (i, 0)),
        ],
        out_shape=[
            jax.ShapeDtypeStruct((N, H), jnp.float32),
            jax.ShapeDtypeStruct((N, H), jnp.float32),
        ],
    )(x, wn, bn_row, degp)


_RE = 2048


def _msg_body(hs_ref, ea_ref, wt_ref, r_ref, se_ref, o_ref):
    hs = hs_ref[...]
    b = jnp.dot(hs, wt_ref[...], preferred_element_type=jnp.float32)
    ear = jnp.dot(ea_ref[...], r_ref[...], preferred_element_type=jnp.float32)
    z = jnp.concatenate([ear * b, hs], axis=1)
    o_ref[...] = jnp.dot(z, se_ref[...], preferred_element_type=jnp.float32)


def _tc_msg(hs, ea, wt, r, se):
    grid = E_PAD // _RE
    return pl.pallas_call(
        _msg_body,
        grid=(grid,),
        in_specs=[
            pl.BlockSpec((_RE, H), lambda i: (i, 0)),
            pl.BlockSpec((_RE, H), lambda i: (i, 0)),
            pl.BlockSpec((H, H * H), lambda i: (0, 0)),
            pl.BlockSpec((H, H * H), lambda i: (0, 0)),
            pl.BlockSpec((H * H + H, H), lambda i: (0, 0)),
        ],
        out_specs=pl.BlockSpec((_RE, H), lambda i: (i, 0)),
        out_shape=jax.ShapeDtypeStruct((E_PAD, H), jnp.float32),
    )(hs, ea, wt, r, se)


def _update_body(h_ref, ap_ref, ig_ref, wr_ref, v_ref, o_ref):
    h = h_ref[...]
    agg = (ap_ref[0] + ap_ref[1]) * ig_ref[...]
    pre = (agg + jnp.dot(h, wr_ref[...], preferred_element_type=jnp.float32))
    out = pre * v_ref[0:1, :] + v_ref[1:2, :]
    o_ref[...] = h + jnp.maximum(out, 0.0)


def _tc_update(h, aggp, ig, wr, vab):
    grid = N // _RN
    return pl.pallas_call(
        _update_body,
        grid=(grid,),
        in_specs=[
            pl.BlockSpec((_RN, H), lambda i: (i, 0)),
            pl.BlockSpec((NC, _RN, H), lambda i: (0, i, 0)),
            pl.BlockSpec((_RN, H), lambda i: (i, 0)),
            pl.BlockSpec((H, H), lambda i: (0, 0)),
            pl.BlockSpec((8, H), lambda i: (0, 0)),
        ],
        out_specs=pl.BlockSpec((_RN, H), lambda i: (i, 0)),
        out_shape=jax.ShapeDtypeStruct((N, H), jnp.float32),
    )(h, aggp, ig, wr, vab)


def _pool_body(h_ref, b_ref, w1_ref, b1_ref, w2_ref, b2_ref, o_ref,
               acc_ref, cnt_ref):
    i = pl.program_id(0)

    @pl.when(i == 0)
    def _():
        acc_ref[...] = jnp.zeros_like(acc_ref)
        cnt_ref[...] = jnp.zeros_like(cnt_ref)

    h = h_ref[...]
    onehot = (b_ref[...] == lax.broadcasted_iota(jnp.int32, (_RN, G), 1))
    onehot = onehot.astype(jnp.float32)
    dn = (((0,), (0,)), ((), ()))
    acc_ref[...] += lax.dot_general(onehot, h, dn,
                                    preferred_element_type=jnp.float32)
    cnt_ref[...] += lax.dot_general(onehot, jnp.ones_like(h), dn,
                                    preferred_element_type=jnp.float32)

    @pl.when(i == pl.num_programs(0) - 1)
    def _():
        pooled = acc_ref[...] / jnp.maximum(cnt_ref[...], 1.0)
        z = jnp.maximum(
            jnp.dot(pooled, w1_ref[...], preferred_element_type=jnp.float32)
            + b1_ref[0:1, :], 0.0)
        o_ref[...] = (
            jnp.dot(z, w2_ref[...], preferred_element_type=jnp.float32)
            + b2_ref[0:1, 0:1]
        )


def _tc_pool(h, batch2, w1, b1_row, w2, b2_row):
    grid = N // _RN
    return pl.pallas_call(
        _pool_body,
        grid=(grid,),
        in_specs=[
            pl.BlockSpec((_RN, H), lambda i: (i, 0)),
            pl.BlockSpec((_RN, 1), lambda i: (i, 0)),
            pl.BlockSpec((H, 64), lambda i: (0, 0)),
            pl.BlockSpec((8, 64), lambda i: (0, 0)),
            pl.BlockSpec((64, 1), lambda i: (0, 0)),
            pl.BlockSpec((8, 8), lambda i: (0, 0)),
        ],
        out_specs=pl.BlockSpec((G, 1), lambda i: (0, 0)),
        out_shape=jax.ShapeDtypeStruct((G, 1), jnp.float32),
        scratch_shapes=[
            pltpu.VMEM((G, H), jnp.float32),
            pltpu.VMEM((G, H), jnp.float32),
        ],
    )(h, batch2, w1, b1_row, w2, b2_row)


def kernel(x, edge_index, edge_attr, batch, Wn, bn_, We0, be0, Wr0, br0, g0,
           bt0, We1, be1, Wr1, br1, g1, bt1, We2, be2, Wr2, br2, g2, bt2,
           W1, b1, W2, b2):
    f32 = jnp.float32
    src = jnp.concatenate(
        [edge_index[0], jnp.zeros((E_PAD - E,), jnp.int32)]).reshape(
            NW * CPW, CK)
    dst = jnp.concatenate(
        [edge_index[1], jnp.full((E_PAD - E,), N, jnp.int32)]).reshape(
            NW * CPW, CK)
    ea = jnp.concatenate(
        [edge_attr, jnp.zeros((E_PAD - E, EDGE_DIM), f32)], axis=0)
    zeros_sh = jnp.zeros((N_SH, H), f32)

    r_mat = jnp.kron(jnp.eye(H, dtype=f32), jnp.ones((1, H), f32))
    s_mat = jnp.kron(jnp.ones((H, 1), f32), jnp.eye(H, dtype=f32))

    scale = 1.0 / jnp.sqrt(1.0 + EPS)
    layers = []
    for (We, be, Wr, br, g, bt) in (
        (We0, be0, Wr0, br0, g0, bt0),
        (We1, be1, Wr1, br1, g1, bt1),
        (We2, be2, Wr2, br2, g2, bt2),
    ):
        wt = We.reshape(H, H, H).transpose(1, 0, 2).reshape(H, H * H)
        se = jnp.concatenate([s_mat, be.reshape(H, H)], axis=0)
        va = g * scale
        vb = br * va + bt
        vab = jnp.zeros((8, H), f32).at[0].set(va).at[1].set(vb)
        layers.append((wt, se, Wr, vab))

    degp = _sc_degree(dst, zeros_sh)
    h, ig = _tc_init(x, Wn, jnp.broadcast_to(bn_, (8, H)), degp)

    for (wt, se, wr, vab) in layers:
        hs = _sc_gather(h, src)
        msg = _tc_msg(hs, ea, wt, r_mat, se)
        aggp = _sc_scatter(msg, dst, zeros_sh)
        h = _tc_update(h, aggp, ig, wr, vab)

    out = _tc_pool(h, batch[:, None], W1, jnp.broadcast_to(b1, (8, 64)), W2,
                   jnp.broadcast_to(b2, (8, 8)))
    return out[:, 0]


# speedup vs baseline: 3.7724x; 1.1399x over previous
import functools

import jax
import jax.numpy as jnp
from jax import lax
from jax.experimental import pallas as pl
from jax.experimental.pallas import tpu as pltpu
from jax.experimental.pallas import tpu_sc as plsc

N = 10000
E = 160000
H = 16
G = 64
NODE_DIM = 128
EDGE_DIM = 16
EPS = 1e-5

NC = 2
NS = 16
NW = NC * NS
CK = 125
CPW = 40
EPW = CPW * CK
KG = 8
N_SH = 10112
NPT = N_SH // NS

_mesh = plsc.VectorSubcoreMesh(core_axis_name="c", subcore_axis_name="s")
_sc_params = pltpu.CompilerParams(use_tc_tiling_on_sc=False)


def _zero_acc(zb, acc, sid):
    def zfill(i, _):
        zb[i] = jnp.zeros((H,), jnp.float32)
        return 0

    lax.fori_loop(0, NPT, zfill, 0)
    pltpu.sync_copy(zb, acc.at[pl.ds(sid * NPT, NPT)])


@functools.partial(
    pl.kernel,
    out_type=[
        jax.ShapeDtypeStruct((E, H), jnp.float32),
        jax.ShapeDtypeStruct((NC, N_SH, H), jnp.float32),
    ],
    mesh=_mesh,
    scratch_types=[
        pltpu.VMEM((CPW, CK), jnp.int32),
        pltpu.VMEM((CPW, CK), jnp.int32),
        pltpu.VMEM((EPW, H), jnp.float32),
        pltpu.VMEM((NPT, H), jnp.float32),
        pltpu.VMEM_SHARED((N_SH, H), jnp.float32),
        pltpu.SemaphoreType.DMA,
        pltpu.SemaphoreType.DMA,
    ],
    compiler_params=_sc_params,
)
def _sc_gather_deg(h_hbm, src_hbm, dst_hbm, hs_hbm, deg_hbm,
                   sidx_v, didx_v, rows_v, zb, acc, sem, dsem):
    cid = lax.axis_index("c")
    sid = lax.axis_index("s")
    wid = cid * NS + sid
    pltpu.sync_copy(src_hbm.at[pl.ds(wid * CPW, CPW)], sidx_v)
    pltpu.sync_copy(dst_hbm.at[pl.ds(wid * CPW, CPW)], didx_v)
    _zero_acc(zb, acc, sid)

    def ofill(i, _):
        rows_v[i] = jnp.ones((H,), jnp.float32)
        return 0

    lax.fori_loop(0, CK, ofill, 0)
    plsc.subcore_barrier()

    def group(g, _):
        base = g * KG
        for t in range(KG):
            pltpu.async_copy(rows_v.at[pl.ds(0, CK)],
                             acc.at[didx_v.at[base + t]], dsem, add=True)
        for t in range(KG):
            pltpu.make_async_copy(rows_v.at[pl.ds(0, CK)],
                                  acc.at[didx_v.at[base + t]], dsem).wait()
        return 0

    lax.fori_loop(0, CPW // KG, group, 0)
    plsc.subcore_barrier()
    pltpu.sync_copy(acc.at[pl.ds(sid * NPT, NPT)],
                    deg_hbm.at[cid, pl.ds(sid * NPT, NPT)])

    def ggroup(g, _):
        base = g * KG
        for t in range(KG):
            pltpu.async_copy(h_hbm.at[sidx_v.at[base + t]],
                             rows_v.at[pl.ds((base + t) * CK, CK)], sem)
        for t in range(KG):
            pltpu.make_async_copy(h_hbm.at[sidx_v.at[base + t]],
                                  rows_v.at[pl.ds((base + t) * CK, CK)],
                                  sem).wait()
        return 0

    lax.fori_loop(0, CPW // KG, ggroup, 0)
    pltpu.sync_copy(rows_v, hs_hbm.at[pl.ds(wid * EPW, EPW)])


@functools.partial(
    pl.kernel,
    out_type=jax.ShapeDtypeStruct((E, H), jnp.float32),
    mesh=_mesh,
    scratch_types=[
        pltpu.VMEM((CPW, CK), jnp.int32),
        pltpu.VMEM((EPW, H), jnp.float32),
        pltpu.SemaphoreType.DMA,
    ],
    compiler_params=_sc_params,
)
def _sc_gather(h_hbm, idx_hbm, out_hbm, idx_v, rows_v, sem):
    wid = lax.axis_index("c") * NS + lax.axis_index("s")
    pltpu.sync_copy(idx_hbm.at[pl.ds(wid * CPW, CPW)], idx_v)

    def group(g, _):
        base = g * KG
        for t in range(KG):
            pltpu.async_copy(h_hbm.at[idx_v.at[base + t]],
                             rows_v.at[pl.ds((base + t) * CK, CK)], sem)
        for t in range(KG):
            pltpu.make_async_copy(h_hbm.at[idx_v.at[base + t]],
                                  rows_v.at[pl.ds((base + t) * CK, CK)],
                                  sem).wait()
        return 0

    lax.fori_loop(0, CPW // KG, group, 0)
    pltpu.sync_copy(rows_v, out_hbm.at[pl.ds(wid * EPW, EPW)])


@functools.partial(
    pl.kernel,
    out_type=jax.ShapeDtypeStruct((NC, N_SH, H), jnp.float32),
    mesh=_mesh,
    scratch_types=[
        pltpu.VMEM((CPW, CK), jnp.int32),
        pltpu.VMEM((EPW, H), jnp.float32),
        pltpu.VMEM((NPT, H), jnp.float32),
        pltpu.VMEM_SHARED((N_SH, H), jnp.float32),
        pltpu.SemaphoreType.DMA,
    ],
    compiler_params=_sc_params,
)
def _sc_scatter(rows_hbm, idx_hbm, out_hbm, idx_v, rows_v, zb, acc, sem):
    cid = lax.axis_index("c")
    sid = lax.axis_index("s")
    wid = cid * NS + sid
    _zero_acc(zb, acc, sid)
    pltpu.sync_copy(idx_hbm.at[pl.ds(wid * CPW, CPW)], idx_v)
    pltpu.sync_copy(rows_hbm.at[pl.ds(wid * EPW, EPW)], rows_v)
    plsc.subcore_barrier()

    def group(g, _):
        base = g * KG
        for t in range(KG):
            pltpu.async_copy(rows_v.at[pl.ds((base + t) * CK, CK)],
                             acc.at[idx_v.at[base + t]], sem, add=True)
        for t in range(KG):
            pltpu.make_async_copy(rows_v.at[pl.ds((base + t) * CK, CK)],
                                  acc.at[idx_v.at[base + t]], sem).wait()
        return 0

    lax.fori_loop(0, CPW // KG, group, 0)
    plsc.subcore_barrier()
    pltpu.sync_copy(acc.at[pl.ds(sid * NPT, NPT)],
                    out_hbm.at[cid, pl.ds(sid * NPT, NPT)])


_RN = 2000


def _init_body(x_ref, wn_ref, bn_ref, h_ref):
    h_ref[...] = (
        jnp.dot(x_ref[...], wn_ref[...], preferred_element_type=jnp.float32)
        + bn_ref[0:1, :]
    )


def _tc_init(x, wn, bn_row):
    grid = N // _RN
    return pl.pallas_call(
        _init_body,
        grid=(grid,),
        in_specs=[
            pl.BlockSpec((_RN, NODE_DIM), lambda i: (i, 0)),
            pl.BlockSpec((NODE_DIM, H), lambda i: (0, 0)),
            pl.BlockSpec((8, H), lambda i: (0, 0)),
        ],
        out_specs=pl.BlockSpec((_RN, H), lambda i: (i, 0)),
        out_shape=jax.ShapeDtypeStruct((N, H), jnp.float32),
    )(x, wn, bn_row)


_RE = 2000


def _msg_body(hs_ref, ea_ref, wt_ref, r_ref, se_ref, o_ref):
    hs = hs_ref[...]
    b = jnp.dot(hs, wt_ref[...], preferred_element_type=jnp.float32)
    ear = jnp.dot(ea_ref[...], r_ref[...], preferred_element_type=jnp.float32)
    z = jnp.concatenate([ear * b, hs], axis=1)
    o_ref[...] = jnp.dot(z, se_ref[...], preferred_element_type=jnp.float32)


def _tc_msg(hs, ea, wt, r, se):
    grid = E // _RE
    return pl.pallas_call(
        _msg_body,
        grid=(grid,),
        in_specs=[
            pl.BlockSpec((_RE, H), lambda i: (i, 0)),
            pl.BlockSpec((_RE, H), lambda i: (i, 0)),
            pl.BlockSpec((H, H * H), lambda i: (0, 0)),
            pl.BlockSpec((H, H * H), lambda i: (0, 0)),
            pl.BlockSpec((H * H + H, H), lambda i: (0, 0)),
        ],
        out_specs=pl.BlockSpec((_RE, H), lambda i: (i, 0)),
        out_shape=jax.ShapeDtypeStruct((E, H), jnp.float32),
    )(hs, ea, wt, r, se)


def _new_h(h_ref, ap_ref, dp_ref, wr_ref, v_ref):
    h = h_ref[...]
    deg = jnp.maximum(dp_ref[0] + dp_ref[1], 1.0)
    agg = (ap_ref[0] + ap_ref[1]) / deg
    pre = (agg + jnp.dot(h, wr_ref[...], preferred_element_type=jnp.float32))
    out = pre * v_ref[0:1, :] + v_ref[1:2, :]
    return h + jnp.maximum(out, 0.0)


def _update_body(h_ref, ap_ref, dp_ref, wr_ref, v_ref, o_ref):
    o_ref[...] = _new_h(h_ref, ap_ref, dp_ref, wr_ref, v_ref)


def _tc_update(h, aggp, degp, wr, vab):
    grid = N // _RN
    return pl.pallas_call(
        _update_body,
        grid=(grid,),
        in_specs=[
            pl.BlockSpec((_RN, H), lambda i: (i, 0)),
            pl.BlockSpec((NC, _RN, H), lambda i: (0, i, 0)),
            pl.BlockSpec((NC, _RN, H), lambda i: (0, i, 0)),
            pl.BlockSpec((H, H), lambda i: (0, 0)),
            pl.BlockSpec((8, H), lambda i: (0, 0)),
        ],
        out_specs=pl.BlockSpec((_RN, H), lambda i: (i, 0)),
        out_shape=jax.ShapeDtypeStruct((N, H), jnp.float32),
    )(h, aggp, degp, wr, vab)


def _final_body(h_ref, ap_ref, dp_ref, wr_ref, v_ref, b_ref,
                w1_ref, b1_ref, w2_ref, b2_ref, o_ref, acc_ref, cnt_ref):
    i = pl.program_id(0)

    @pl.when(i == 0)
    def _():
        acc_ref[...] = jnp.zeros_like(acc_ref)
        cnt_ref[...] = jnp.zeros_like(cnt_ref)

    hn = _new_h(h_ref, ap_ref, dp_ref, wr_ref, v_ref)
    onehot = (b_ref[...] == lax.broadcasted_iota(jnp.int32, (_RN, G), 1))
    onehot = onehot.astype(jnp.float32)
    dn = (((0,), (0,)), ((), ()))
    acc_ref[...] += lax.dot_general(onehot, hn, dn,
                                    preferred_element_type=jnp.float32)
    cnt_ref[...] += lax.dot_general(onehot, jnp.ones_like(hn), dn,
                                    preferred_element_type=jnp.float32)

    @pl.when(i == pl.num_programs(0) - 1)
    def _():
        pooled = acc_ref[...] / jnp.maximum(cnt_ref[...], 1.0)
        z = jnp.maximum(
            jnp.dot(pooled, w1_ref[...], preferred_element_type=jnp.float32)
            + b1_ref[0:1, :], 0.0)
        o_ref[...] = (
            jnp.dot(z, w2_ref[...], preferred_element_type=jnp.float32)
            + b2_ref[0:1, 0:1]
        )


def _tc_final(h, aggp, degp, wr, vab, batch2, w1, b1_row, w2, b2_row):
    grid = N // _RN
    return pl.pallas_call(
        _final_body,
        grid=(grid,),
        in_specs=[
            pl.BlockSpec((_RN, H), lambda i: (i, 0)),
            pl.BlockSpec((NC, _RN, H), lambda i: (0, i, 0)),
            pl.BlockSpec((NC, _RN, H), lambda i: (0, i, 0)),
            pl.BlockSpec((H, H), lambda i: (0, 0)),
            pl.BlockSpec((8, H), lambda i: (0, 0)),
            pl.BlockSpec((_RN, 1), lambda i: (i, 0)),
            pl.BlockSpec((H, 64), lambda i: (0, 0)),
            pl.BlockSpec((8, 64), lambda i: (0, 0)),
            pl.BlockSpec((64, 1), lambda i: (0, 0)),
            pl.BlockSpec((8, 8), lambda i: (0, 0)),
        ],
        out_specs=pl.BlockSpec((G, 1), lambda i: (0, 0)),
        out_shape=jax.ShapeDtypeStruct((G, 1), jnp.float32),
        scratch_shapes=[
            pltpu.VMEM((G, H), jnp.float32),
            pltpu.VMEM((G, H), jnp.float32),
        ],
    )(h, aggp, degp, wr, vab, batch2, w1, b1_row, w2, b2_row)


def kernel(x, edge_index, edge_attr, batch, Wn, bn_, We0, be0, Wr0, br0, g0,
           bt0, We1, be1, Wr1, br1, g1, bt1, We2, be2, Wr2, br2, g2, bt2,
           W1, b1, W2, b2):
    f32 = jnp.float32
    src = edge_index[0].reshape(NW * CPW, CK)
    dst = edge_index[1].reshape(NW * CPW, CK)

    r_mat = jnp.kron(jnp.eye(H, dtype=f32), jnp.ones((1, H), f32))
    s_mat = jnp.kron(jnp.ones((H, 1), f32), jnp.eye(H, dtype=f32))

    scale = 1.0 / jnp.sqrt(1.0 + EPS)
    layers = []
    for (We, be, Wr, br, g, bt) in (
        (We0, be0, Wr0, br0, g0, bt0),
        (We1, be1, Wr1, br1, g1, bt1),
        (We2, be2, Wr2, br2, g2, bt2),
    ):
        wt = We.reshape(H, H, H).transpose(1, 0, 2).reshape(H, H * H)
        se = jnp.concatenate([s_mat, be.reshape(H, H)], axis=0)
        va = g * scale
        vb = br * va + bt
        vab = jnp.zeros((8, H), f32).at[0].set(va).at[1].set(vb)
        layers.append((wt, se, Wr, vab))

    h = _tc_init(x, Wn, jnp.broadcast_to(bn_, (8, H)))
    hs, degp = _sc_gather_deg(h, src, dst)

    for li, (wt, se, wr, vab) in enumerate(layers):
        msg = _tc_msg(hs, edge_attr, wt, r_mat, se)
        aggp = _sc_scatter(msg, dst)
        if li < 2:
            h = _tc_update(h, aggp, degp, wr, vab)
            hs = _sc_gather(h, src)
        else:
            out = _tc_final(h, aggp, degp, wr, vab, batch[:, None],
                            W1, jnp.broadcast_to(b1, (8, 64)), W2,
                            jnp.broadcast_to(b2, (8, 8)))
    return out[:, 0]


# speedup vs baseline: 7.2682x; 1.9267x over previous
import functools

import jax
import jax.numpy as jnp
from jax import lax
from jax.experimental import pallas as pl
from jax.experimental.pallas import tpu as pltpu
from jax.experimental.pallas import tpu_sc as plsc

N = 10000
E = 160000
H = 16
G = 64
NODE_DIM = 128
EDGE_DIM = 16
EPS = 1e-5

NC = 2
NS = 16
NW = NC * NS
CK = 125
CPW = 40
EPW = CPW * CK
KG = 8
N_SH = 10112
NPT = N_SH // NS

_mesh = plsc.VectorSubcoreMesh(core_axis_name="c", subcore_axis_name="s")
_sc_params = pltpu.CompilerParams(use_tc_tiling_on_sc=False)


def _zero_acc(zb, acc, sid):
    def zfill(i, _):
        zb[i] = jnp.zeros((H,), jnp.float32)
        return 0

    lax.fori_loop(0, NPT, zfill, 0)
    pltpu.sync_copy(zb, acc.at[pl.ds(sid * NPT, NPT)])


@functools.partial(
    pl.kernel,
    out_type=[
        jax.ShapeDtypeStruct((E, H), jnp.float32),
        jax.ShapeDtypeStruct((NC, N_SH, H), jnp.float32),
    ],
    mesh=_mesh,
    scratch_types=[
        pltpu.VMEM((CPW, CK), jnp.int32),
        pltpu.VMEM((CPW, CK), jnp.int32),
        pltpu.VMEM((EPW, H), jnp.float32),
        pltpu.VMEM((NPT, H), jnp.float32),
        pltpu.VMEM_SHARED((N_SH, H), jnp.float32),
        pltpu.SemaphoreType.DMA,
        pltpu.SemaphoreType.DMA,
    ],
    compiler_params=_sc_params,
)
def _sc_gather_deg(h_hbm, src_hbm, dst_hbm, hs_hbm, deg_hbm,
                   sidx_v, didx_v, rows_v, zb, acc, sem, dsem):
    cid = lax.axis_index("c")
    sid = lax.axis_index("s")
    wid = cid * NS + sid
    pltpu.sync_copy(src_hbm.at[pl.ds(wid * CPW, CPW)], sidx_v)
    pltpu.sync_copy(dst_hbm.at[pl.ds(wid * CPW, CPW)], didx_v)
    _zero_acc(zb, acc, sid)

    def ofill(i, _):
        rows_v[i] = jnp.ones((H,), jnp.float32)
        return 0

    lax.fori_loop(0, CK, ofill, 0)
    plsc.subcore_barrier()

    def group(g, _):
        base = g * KG
        for t in range(KG):
            pltpu.async_copy(rows_v.at[pl.ds(0, CK)],
                             acc.at[didx_v.at[base + t]], dsem, add=True)
        for t in range(KG):
            pltpu.make_async_copy(rows_v.at[pl.ds(0, CK)],
                                  acc.at[didx_v.at[base + t]], dsem).wait()
        return 0

    lax.fori_loop(0, CPW // KG, group, 0)
    plsc.subcore_barrier()
    pltpu.sync_copy(acc.at[pl.ds(sid * NPT, NPT)],
                    deg_hbm.at[cid, pl.ds(sid * NPT, NPT)])

    def ggroup(g, _):
        base = g * KG
        for t in range(KG):
            pltpu.async_copy(h_hbm.at[sidx_v.at[base + t]],
                             rows_v.at[pl.ds((base + t) * CK, CK)], sem)
        for t in range(KG):
            pltpu.make_async_copy(h_hbm.at[sidx_v.at[base + t]],
                                  rows_v.at[pl.ds((base + t) * CK, CK)],
                                  sem).wait()
        return 0

    lax.fori_loop(0, CPW // KG, ggroup, 0)
    pltpu.sync_copy(rows_v, hs_hbm.at[pl.ds(wid * EPW, EPW)])


@functools.partial(
    pl.kernel,
    out_type=jax.ShapeDtypeStruct((E, H), jnp.float32),
    mesh=_mesh,
    scratch_types=[
        pltpu.VMEM((CPW, CK), jnp.int32),
        pltpu.VMEM((EPW, H), jnp.float32),
        pltpu.SemaphoreType.DMA,
    ],
    compiler_params=_sc_params,
)
def _sc_gather(h_hbm, idx_hbm, out_hbm, idx_v, rows_v, sem):
    wid = lax.axis_index("c") * NS + lax.axis_index("s")
    pltpu.sync_copy(idx_hbm.at[pl.ds(wid * CPW, CPW)], idx_v)

    def group(g, _):
        base = g * KG
        for t in range(KG):
            pltpu.async_copy(h_hbm.at[idx_v.at[base + t]],
                             rows_v.at[pl.ds((base + t) * CK, CK)], sem)
        for t in range(KG):
            pltpu.make_async_copy(h_hbm.at[idx_v.at[base + t]],
                                  rows_v.at[pl.ds((base + t) * CK, CK)],
                                  sem).wait()
        return 0

    lax.fori_loop(0, CPW // KG, group, 0)
    pltpu.sync_copy(rows_v, out_hbm.at[pl.ds(wid * EPW, EPW)])


@functools.partial(
    pl.kernel,
    out_type=jax.ShapeDtypeStruct((NC, N_SH, H), jnp.float32),
    mesh=_mesh,
    scratch_types=[
        pltpu.VMEM((CPW, CK), jnp.int32),
        pltpu.VMEM((EPW, H), jnp.float32),
        pltpu.VMEM((NPT, H), jnp.float32),
        pltpu.VMEM_SHARED((N_SH, H), jnp.float32),
        pltpu.SemaphoreType.DMA,
    ],
    compiler_params=_sc_params,
)
def _sc_scatter(rows_hbm, idx_hbm, out_hbm, idx_v, rows_v, zb, acc, sem):
    cid = lax.axis_index("c")
    sid = lax.axis_index("s")
    wid = cid * NS + sid
    _zero_acc(zb, acc, sid)
    pltpu.sync_copy(idx_hbm.at[pl.ds(wid * CPW, CPW)], idx_v)
    pltpu.sync_copy(rows_hbm.at[pl.ds(wid * EPW, EPW)], rows_v)
    plsc.subcore_barrier()

    def group(g, _):
        base = g * KG
        for t in range(KG):
            pltpu.async_copy(rows_v.at[pl.ds((base + t) * CK, CK)],
                             acc.at[idx_v.at[base + t]], sem, add=True)
        for t in range(KG):
            pltpu.make_async_copy(rows_v.at[pl.ds((base + t) * CK, CK)],
                                  acc.at[idx_v.at[base + t]], sem).wait()
        return 0

    lax.fori_loop(0, CPW // KG, group, 0)
    plsc.subcore_barrier()
    pltpu.sync_copy(acc.at[pl.ds(sid * NPT, NPT)],
                    out_hbm.at[cid, pl.ds(sid * NPT, NPT)])


_NP = N // 8
_NSP = N_SH // 8
_RE = 3200
_REP = _RE // 8
_EP = E // 8


def _init_body(x_ref, wn_ref, bn_ref, h_ref):
    h_ref[...] = (
        jnp.dot(x_ref[...], wn_ref[...], preferred_element_type=jnp.float32)
        + bn_ref[0:1, :]
    )


def _tc_init(x_p, wn_big, bn_row):
    return pl.pallas_call(
        _init_body,
        out_shape=jax.ShapeDtypeStruct((_NP, 128), jnp.float32),
    )(x_p, wn_big, bn_row)


def _msg_body(hs_ref, ea_ref, wt_ref, r_ref, s_ref, bm_ref, o_ref):
    hs = hs_ref[...]
    b = jnp.dot(hs, wt_ref[...], preferred_element_type=jnp.float32)
    ear = jnp.dot(ea_ref[...], r_ref[...], preferred_element_type=jnp.float32)
    o_ref[...] = (
        jnp.dot(ear * b, s_ref[...], preferred_element_type=jnp.float32)
        + jnp.dot(hs, bm_ref[...], preferred_element_type=jnp.float32)
    )


def _tc_msg(hs_p, ea_p, wt_big, r_big, s_big, bm_big):
    grid = E // _RE
    return pl.pallas_call(
        _msg_body,
        grid=(grid,),
        in_specs=[
            pl.BlockSpec((_REP, 128), lambda i: (i, 0)),
            pl.BlockSpec((_REP, 128), lambda i: (i, 0)),
            pl.BlockSpec((128, 8 * H * H), lambda i: (0, 0)),
            pl.BlockSpec((128, 8 * H * H), lambda i: (0, 0)),
            pl.BlockSpec((8 * H * H, 128), lambda i: (0, 0)),
            pl.BlockSpec((128, 128), lambda i: (0, 0)),
        ],
        out_specs=pl.BlockSpec((_REP, 128), lambda i: (i, 0)),
        out_shape=jax.ShapeDtypeStruct((_EP, 128), jnp.float32),
    )(hs_p, ea_p, wt_big, r_big, s_big, bm_big)


def _new_h(h_ref, ap_ref, dp_ref, wr_ref, v_ref):
    h = h_ref[...]
    deg = jnp.maximum((dp_ref[0] + dp_ref[1])[:_NP], 1.0)
    agg = (ap_ref[0] + ap_ref[1])[:_NP] / deg
    pre = (agg + jnp.dot(h, wr_ref[...], preferred_element_type=jnp.float32))
    out = pre * v_ref[0:1, :] + v_ref[1:2, :]
    return h + jnp.maximum(out, 0.0)


def _update_body(h_ref, ap_ref, dp_ref, wr_ref, v_ref, o_ref):
    o_ref[...] = _new_h(h_ref, ap_ref, dp_ref, wr_ref, v_ref)


def _tc_update(h_p, aggp_p, degp_p, wr_big, vab):
    return pl.pallas_call(
        _update_body,
        out_shape=jax.ShapeDtypeStruct((_NP, 128), jnp.float32),
    )(h_p, aggp_p, degp_p, wr_big, vab)


def _final_body(h_ref, ap_ref, dp_ref, wr_ref, v_ref, b_ref,
                w1_ref, b1_ref, w2_ref, b2_ref, o_ref):
    hn = _new_h(h_ref, ap_ref, dp_ref, wr_ref, v_ref)
    dn = (((0,), (0,)), ((), ()))
    acc = jnp.zeros((G, H), jnp.float32)
    cnt = jnp.zeros((G, H), jnp.float32)
    iota = lax.broadcasted_iota(jnp.int32, (_NP, G), 1)
    for e in range(8):
        onehot = (b_ref[:, e:e + 1] == iota).astype(jnp.float32)
        hslice = hn[:, e * H:(e + 1) * H]
        acc = acc + lax.dot_general(onehot, hslice, dn,
                                    preferred_element_type=jnp.float32)
        cnt = cnt + lax.dot_general(onehot, jnp.ones_like(hslice), dn,
                                    preferred_element_type=jnp.float32)
    pooled = acc / jnp.maximum(cnt, 1.0)
    z = jnp.maximum(
        jnp.dot(pooled, w1_ref[...], preferred_element_type=jnp.float32)
        + b1_ref[0:1, :], 0.0)
    o_ref[...] = (
        jnp.dot(z, w2_ref[...], preferred_element_type=jnp.float32)
        + b2_ref[0:1, 0:1]
    )


def _tc_final(h_p, aggp_p, degp_p, wr_big, vab, batch_p, w1, b1_row, w2,
              b2_row):
    return pl.pallas_call(
        _final_body,
        out_shape=jax.ShapeDtypeStruct((G, 1), jnp.float32),
    )(h_p, aggp_p, degp_p, wr_big, vab, batch_p, w1, b1_row, w2, b2_row)


def kernel(x, edge_index, edge_attr, batch, Wn, bn_, We0, be0, Wr0, br0, g0,
           bt0, We1, be1, Wr1, br1, g1, bt1, We2, be2, Wr2, br2, g2, bt2,
           W1, b1, W2, b2):
    f32 = jnp.float32
    src = edge_index[0].reshape(NW * CPW, CK)
    dst = edge_index[1].reshape(NW * CPW, CK)
    eye8 = jnp.eye(8, dtype=f32)

    r_mat = jnp.kron(jnp.eye(H, dtype=f32), jnp.ones((1, H), f32))
    s_mat = jnp.kron(jnp.ones((H, 1), f32), jnp.eye(H, dtype=f32))
    r_big = jnp.kron(eye8, r_mat)
    s_big = jnp.kron(eye8, s_mat)

    scale = 1.0 / jnp.sqrt(1.0 + EPS)
    layers = []
    for (We, be, Wr, br, g, bt) in (
        (We0, be0, Wr0, br0, g0, bt0),
        (We1, be1, Wr1, br1, g1, bt1),
        (We2, be2, Wr2, br2, g2, bt2),
    ):
        wt = We.reshape(H, H, H).transpose(1, 0, 2).reshape(H, H * H)
        wt_big = jnp.kron(eye8, wt)
        bm_big = jnp.kron(eye8, be.reshape(H, H))
        wr_big = jnp.kron(eye8, Wr)
        va = jnp.tile(g * scale, 8)
        vb = jnp.tile(br * g * scale + bt, 8)
        vab = jnp.zeros((8, 128), f32).at[0].set(va).at[1].set(vb)
        layers.append((wt_big, s_big, bm_big, wr_big, vab))

    h_p = _tc_init(x.reshape(_NP, 8 * NODE_DIM), jnp.kron(eye8, Wn),
                   jnp.broadcast_to(jnp.tile(bn_, 8), (8, 128)))
    ea_p = edge_attr.reshape(_EP, 128)
    hs, degp = _sc_gather_deg(h_p.reshape(N, H), src, dst)
    degp_p = degp.reshape(NC, _NSP, 128)

    for li, (wt_big, s_big_l, bm_big, wr_big, vab) in enumerate(layers):
        msg_p = _tc_msg(hs.reshape(_EP, 128), ea_p, wt_big, r_big, s_big_l,
                        bm_big)
        aggp = _sc_scatter(msg_p.reshape(E, H), dst)
        aggp_p = aggp.reshape(NC, _NSP, 128)
        if li < 2:
            h_p = _tc_update(h_p, aggp_p, degp_p, wr_big, vab)
            hs = _sc_gather(h_p.reshape(N, H), src)
        else:
            out = _tc_final(h_p, aggp_p, degp_p, wr_big, vab,
                            batch.reshape(_NP, 8), W1,
                            jnp.broadcast_to(b1, (8, 64)), W2,
                            jnp.broadcast_to(b2, (8, 8)))
    return out[:, 0]
